# baseline (speedup 1.0000x reference)
import jax
import jax.numpy as jnp
from jax.experimental import pallas as pl
from jax.experimental.pallas import tpu as pltpu

B, D, K = 1024, 512, 32
BT = 256
NTH = 511
F0 = D // 2


def _circuit_kernel(x_ref, mu_ref, *refs):
    th_refs = refs[:9]
    out_ref, w_scr, mup_scr, xb_scr = refs[9:]

    @pl.when(pl.program_id(0) == 0)
    def _prep():
        off = 0
        for th_ref in th_refs:
            th = th_ref[...]
            f = th.shape[0]
            u = jnp.exp(th)
            ones = jnp.ones((f, K, 1), jnp.float32)
            z = jax.lax.dot_general(
                u, ones,
                dimension_numbers=(((2,), (1,)), ((0,), (0,))),
                preferred_element_type=jnp.float32,
            )
            w_scr[off:off + f] = u * (1.0 / z)
            off += f
        mu = mu_ref[...].reshape(F0, 2, K)
        mup_scr[:, 0:2, :] = mu
        musq = mu * mu
        mup_scr[:, 2, :] = -0.5 * (musq[:, 0] + musq[:, 1])
        xb_scr[:, 2, :] = jnp.ones((F0, BT), jnp.float32)

    xt = jnp.transpose(x_ref[...].reshape(BT, D))
    xsq = (xt * xt).reshape(F0, 2, BT)
    bv = -0.5 * (xsq[:, 0] + xsq[:, 1])
    xb_scr[:, 0:2, :] = xt.reshape(F0, 2, BT)
    prod = jax.lax.dot_general(
        mup_scr[...], xb_scr[...],
        dimension_numbers=(((1,), (1,)), ((0,), (0,))),
        preferred_element_type=jnp.float32,
    )
    m = jnp.max(prod, axis=1, keepdims=True)
    e = jnp.exp(prod - m)
    m = m + bv[:, None, :]

    off = 0
    f = F0
    while True:
        s = jax.lax.dot_general(
            w_scr[off:off + f], e,
            dimension_numbers=(((2,), (1,)), ((0,), (0,))),
            preferred_element_type=jnp.float32,
        )
        if f == 1:
            res = jnp.transpose(m[0] + jnp.log(s[0]))
            out_ref[...] = res[:, None, :]
            break
        off += f
        f //= 2
        sp = s.reshape(f, 2, K, BT)
        sp = sp[:, 0] * sp[:, 1]
        mp = m.reshape(f, 2, 1, BT)
        m = mp[:, 0] + mp[:, 1]
        ones = jnp.ones((f, 1, K), jnp.float32)
        t = jax.lax.dot_general(
            ones, sp,
            dimension_numbers=(((2,), (1,)), ((0,), (0,))),
            preferred_element_type=jnp.float32,
        )
        e = sp * (1.0 / t)
        m = m + jnp.log(t)


@jax.jit
def kernel(x, mu, theta0, theta1, theta2, theta3, theta4, theta5, theta6,
           theta7, theta8):
    thetas = [theta0, theta1, theta2, theta3, theta4, theta5, theta6,
              theta7, theta8]

    th_specs = [
        pl.BlockSpec((max(F0 >> j, 1), K, K), lambda i: (0, 0, 0))
        for j in range(9)
    ]
    out = pl.pallas_call(
        _circuit_kernel,
        grid=(B // BT,),
        in_specs=[
            pl.BlockSpec((BT, 1, D), lambda i: (i, 0, 0)),
            pl.BlockSpec((D, 1, 1, K), lambda i: (0, 0, 0, 0)),
        ] + th_specs,
        out_specs=pl.BlockSpec((BT, 1, K), lambda i: (i, 0, 0)),
        out_shape=jax.ShapeDtypeStruct((B, 1, K), jnp.float32),
        scratch_shapes=[
            pltpu.VMEM((NTH, K, K), jnp.float32),
            pltpu.VMEM((F0, 3, K), jnp.float32),
            pltpu.VMEM((F0, 3, BT), jnp.float32),
        ],
        compiler_params=pltpu.CompilerParams(
            dimension_semantics=("arbitrary",),
        ),
    )(x, mu, *thetas)
    return out


# speedup vs baseline: 1.0960x; 1.0960x over previous
import jax
import jax.numpy as jnp
from jax.experimental import pallas as pl
from jax.experimental.pallas import tpu as pltpu

B, D, K = 1024, 512, 32
BT = 512
NTH = 511
F0 = D // 2


def _circuit_kernel(x_ref, mu_ref, *refs):
    th_refs = refs[:9]
    out_ref, w_scr, mup_scr, xb_scr = refs[9:]

    @pl.when(pl.program_id(0) == 0)
    def _prep():
        off = 0
        for th_ref in th_refs:
            th = th_ref[...]
            f = th.shape[0]
            u = jnp.exp(th)
            ones = jnp.ones((f, K, 1), jnp.float32)
            z = jax.lax.dot_general(
                u, ones,
                dimension_numbers=(((2,), (1,)), ((0,), (0,))),
                preferred_element_type=jnp.float32,
            )
            w_scr[off:off + f] = (u * (1.0 / z)).astype(jnp.bfloat16)
            off += f
        mu = mu_ref[...].reshape(F0, 2, K)
        mup_scr[:, 0:2, :] = mu
        musq = mu * mu
        mup_scr[:, 2, :] = -0.5 * (musq[:, 0] + musq[:, 1])
        xb_scr[:, 2, :] = jnp.ones((F0, BT), jnp.float32)

    xt = jnp.transpose(x_ref[...].reshape(BT, D))
    xsq = (xt * xt).reshape(F0, 2, BT)
    bv = -0.5 * (xsq[:, 0] + xsq[:, 1])
    xb_scr[:, 0:2, :] = xt.reshape(F0, 2, BT)
    prod = jax.lax.dot_general(
        mup_scr[...], xb_scr[...],
        dimension_numbers=(((1,), (1,)), ((0,), (0,))),
        preferred_element_type=jnp.float32,
    )
    m = jnp.max(prod, axis=1, keepdims=True)
    e = jnp.exp(prod - m).astype(jnp.bfloat16)
    m = m + bv[:, None, :]

    off = 0
    f = F0
    while True:
        s = jax.lax.dot_general(
            w_scr[off:off + f], e,
            dimension_numbers=(((2,), (1,)), ((0,), (0,))),
            preferred_element_type=jnp.float32,
        )
        if f == 1:
            res = jnp.transpose(m[0] + jnp.log(s[0]))
            out_ref[...] = res[:, None, :]
            break
        off += f
        f //= 2
        sp = s.reshape(f, 2, K, BT)
        sp = sp[:, 0] * sp[:, 1]
        mp = m.reshape(f, 2, 1, BT)
        m = mp[:, 0] + mp[:, 1]
        ones = jnp.ones((f, 1, K), jnp.float32)
        t = jax.lax.dot_general(
            ones, sp,
            dimension_numbers=(((2,), (1,)), ((0,), (0,))),
            preferred_element_type=jnp.float32,
        )
        e = (sp * (1.0 / t)).astype(jnp.bfloat16)
        m = m + jnp.log(t)


@jax.jit
def kernel(x, mu, theta0, theta1, theta2, theta3, theta4, theta5, theta6,
           theta7, theta8):
    thetas = [theta0, theta1, theta2, theta3, theta4, theta5, theta6,
              theta7, theta8]

    th_specs = [
        pl.BlockSpec((max(F0 >> j, 1), K, K), lambda i: (0, 0, 0))
        for j in range(9)
    ]
    out = pl.pallas_call(
        _circuit_kernel,
        grid=(B // BT,),
        in_specs=[
            pl.BlockSpec((BT, 1, D), lambda i: (i, 0, 0)),
            pl.BlockSpec((D, 1, 1, K), lambda i: (0, 0, 0, 0)),
        ] + th_specs,
        out_specs=pl.BlockSpec((BT, 1, K), lambda i: (i, 0, 0)),
        out_shape=jax.ShapeDtypeStruct((B, 1, K), jnp.float32),
        scratch_shapes=[
            pltpu.VMEM((NTH, K, K), jnp.bfloat16),
            pltpu.VMEM((F0, 3, K), jnp.float32),
            pltpu.VMEM((F0, 3, BT), jnp.float32),
        ],
        compiler_params=pltpu.CompilerParams(
            dimension_semantics=("arbitrary",),
        ),
    )(x, mu, *thetas)
    return out
